# Initial kernel scaffold; baseline (speedup 1.0000x reference)
import jax
import jax.numpy as jnp
from jax.experimental import pallas as pl


def kernel(inputs, indexes, targets, weight, features, labels, labels_1):
    raise NotImplementedError("write your pallas kernel here")


# speedup vs baseline: 3.0302x; 3.0302x over previous
import functools

import jax
import jax.numpy as jnp
from jax import lax
from jax.experimental import pallas as pl
from jax.experimental.pallas import tpu as pltpu
from jax.experimental.pallas import tpu_sc as plsc

NF = 128
HF = NF // 2
NS = 100000
NC = 500
CPAD = 512
B = 128
TEMP = 0.05
NTILES = 32
NPAIRS = 16
CHUNK = 128
NFULL = NS // CHUNK
REM_BASE = NFULL * CHUNK
REM = NS - REM_BASE
EXTRA_PAIRS = NFULL - (NFULL // NPAIRS) * NPAIRS
BASE_CHUNKS = NFULL // NPAIRS

_mesh = plsc.VectorSubcoreMesh(core_axis_name="c", subcore_axis_name="s")


@functools.partial(
    pl.kernel,
    out_type=[
        jax.ShapeDtypeStruct((NTILES, CPAD, HF), jnp.float32),
        jax.ShapeDtypeStruct((NTILES, CPAD, 16), jnp.float32),
    ],
    mesh=_mesh,
    compiler_params=pltpu.CompilerParams(needs_layout_passes=False,
                                         use_tc_tiling_on_sc=False),
    scratch_types=[
        pltpu.VMEM((CPAD, HF), jnp.float32),
        pltpu.VMEM((CPAD, 16), jnp.float32),
        pltpu.VMEM((CHUNK, HF), jnp.float32),
        pltpu.VMEM((CHUNK,), jnp.int32),
        pltpu.VMEM((CHUNK,), jnp.int32),
        pltpu.SemaphoreType.DMA,
        pltpu.SemaphoreType.DMA,
        pltpu.SemaphoreType.DMA,
    ],
)
def _sc_segment_sum(feat_hbm, lab_hbm, lab1_hbm, outg_hbm, outc_hbm,
                    acc, cnt, fbuf, lbuf, l1buf,
                    semf, seml, sem1):
    wid = lax.axis_index("s") * 2 + lax.axis_index("c")
    pair = wid // 2
    half = wid % 2
    col0 = half * HF
    zero16 = jnp.zeros((16,), jnp.float32)
    e0 = (lax.iota(jnp.int32, 16) == 0).astype(jnp.float32)

    @pl.loop(0, CPAD)
    def _zero(r):
        for k in range(HF // 16):
            acc[r, pl.ds(k * 16, 16)] = zero16
        cnt[r, :] = zero16

    def process_chunk(base, nrows):
        cpf = pltpu.async_copy(
            feat_hbm.at[pl.ds(base, nrows), pl.ds(col0, HF)],
            fbuf.at[pl.ds(0, nrows)], semf)
        cpl = pltpu.async_copy(lab_hbm.at[pl.ds(base, nrows)],
                               lbuf.at[pl.ds(0, nrows)], seml)
        cp1 = pltpu.async_copy(lab1_hbm.at[pl.ds(base, nrows)],
                               l1buf.at[pl.ds(0, nrows)], sem1)
        cpl.wait()
        cp1.wait()
        cpf.wait()

        @pl.loop(0, nrows // 16)
        def _grp(j):
            lab_vec = lbuf[pl.ds(j * 16, 16)]
            l1_vec = l1buf[pl.ds(j * 16, 16)]
            labp_vec = jnp.where(l1_vec <= 0, lab_vec, CPAD - 1)
            for l in range(16):
                labp = labp_vec[l]
                for k in range(HF // 16):
                    plsc.addupdate(acc.at[labp, pl.ds(k * 16, 16)],
                                   fbuf[j * 16 + l, pl.ds(k * 16, 16)])
                plsc.addupdate(cnt.at[labp, :], e0)

    n_chunks = jnp.where(pair < EXTRA_PAIRS, BASE_CHUNKS + 1, BASE_CHUNKS)

    @pl.loop(0, n_chunks)
    def _chunks(i):
        process_chunk((i * NPAIRS + pair) * CHUNK, CHUNK)

    @pl.when(pair == EXTRA_PAIRS)
    def _rem():
        process_chunk(REM_BASE, REM)

    pltpu.async_copy(acc, outg_hbm.at[wid], semf).wait()
    pltpu.async_copy(cnt, outc_hbm.at[wid], semf).wait()


def _tc_finish_body(pg_ref, pc_ref, in_ref, tgt_ref, w_ref, out_ref):
    cnt = jnp.sum(pc_ref[...], axis=0)
    nums = jnp.sum(cnt, axis=1, keepdims=True) * 0.5
    denom = jnp.where(nums > 0, nums, 1.0)
    inv = 1.0 / denom

    pg = jnp.sum(pg_ref[...], axis=0)
    g0 = pg[0] * inv
    g1 = pg[1] * inv
    dn = (((1,), (1,)), ((), ()))
    vec = (lax.dot_general(in_ref[:, :HF], g0, dn,
                           precision=lax.Precision.HIGHEST)
           + lax.dot_general(in_ref[:, HF:], g1, dn,
                             precision=lax.Precision.HIGHEST)) * (1.0 / TEMP)

    ones_col = jnp.ones((B, 1), jnp.float32)
    nums_b = lax.dot_general(ones_col, nums, dn,
                             precision=lax.Precision.HIGHEST)
    col_id = lax.broadcasted_iota(jnp.int32, (B, CPAD), 1)
    m = jnp.logical_and(col_id < NC, nums_b > 0)
    mf = m.astype(jnp.float32)

    vecm = jnp.where(m, vec, 0.0)
    exps = jnp.exp(vecm) * mf
    sums = jnp.sum(exps, axis=1, keepdims=True) + 1e-6
    masked_sim = exps / sums
    log_probs = jnp.log(masked_sim + 1e-6)
    lossmat = -tgt_ref[...] * log_probs * w_ref[...]
    total = jnp.sum(lossmat) * (1.0 / B)
    out_ref[...] = jnp.full((8, 128), total, jnp.float32)


_tc_finish = pl.pallas_call(
    _tc_finish_body,
    out_shape=jax.ShapeDtypeStruct((8, 128), jnp.float32),
)


@jax.jit
def kernel(inputs, indexes, targets, weight, features, labels, labels_1):
    del indexes
    partg, partc = _sc_segment_sum(features, labels.astype(jnp.int32),
                                   labels_1.astype(jnp.int32))
    partg = partg.reshape(NPAIRS, 2, CPAD, HF)
    tgt = jnp.pad(targets, ((0, 0), (0, CPAD - NC)))
    out = _tc_finish(partg, partc, inputs, tgt, weight.reshape(B, 1))
    return out[0, 0]


# speedup vs baseline: 5.9391x; 1.9600x over previous
import functools

import jax
import jax.numpy as jnp
from jax import lax
from jax.experimental import pallas as pl
from jax.experimental.pallas import tpu as pltpu
from jax.experimental.pallas import tpu_sc as plsc

NF = 128
HF = NF // 2
NS = 100000
NC = 500
CPAD = 512
B = 128
TEMP = 0.05
NTILES = 32
NPAIRS = 16
CHUNK = 384
NCHUNKS = 16
MAIN_ROWS = NPAIRS * NCHUNKS * CHUNK
TAIL_A = 112
TAIL_B = 16

_mesh = plsc.VectorSubcoreMesh(core_axis_name="c", subcore_axis_name="s")


@functools.partial(
    pl.kernel,
    out_type=[
        jax.ShapeDtypeStruct((NTILES, CPAD, HF), jnp.float32),
        jax.ShapeDtypeStruct((NTILES, CPAD, 16), jnp.float32),
    ],
    mesh=_mesh,
    compiler_params=pltpu.CompilerParams(needs_layout_passes=False,
                                         use_tc_tiling_on_sc=False),
    scratch_types=[
        pltpu.VMEM((CPAD, HF), jnp.float32),
        pltpu.VMEM((CPAD, 16), jnp.float32),
        pltpu.VMEM((2, CHUNK, HF), jnp.float32),
        pltpu.VMEM((2, CHUNK), jnp.int32),
        pltpu.VMEM((2, CHUNK), jnp.int32),
        pltpu.SemaphoreType.DMA,
        pltpu.SemaphoreType.DMA,
        pltpu.SemaphoreType.DMA,
        pltpu.SemaphoreType.DMA,
        pltpu.SemaphoreType.DMA,
        pltpu.SemaphoreType.DMA,
    ],
)
def _sc_segment_sum(feat_hbm, lab_hbm, lab1_hbm, outg_hbm, outc_hbm,
                    acc, cnt, fbuf, lbuf, l1buf,
                    semf0, seml0, sem10, semf1, seml1, sem11):
    wid = lax.axis_index("s") * 2 + lax.axis_index("c")
    pair = wid // 2
    half = wid % 2
    col0 = half * HF
    zero16 = jnp.zeros((16,), jnp.float32)
    e0 = (lax.iota(jnp.int32, 16) == 0).astype(jnp.float32)
    sems = [(semf0, seml0, sem10), (semf1, seml1, sem11)]

    @pl.loop(0, CPAD)
    def _zero(r):
        for k in range(HF // 16):
            acc[r, pl.ds(k * 16, 16)] = zero16
        cnt[r, :] = zero16

    def issue(base, nrows, b):
        semf, seml, sem1 = sems[b]
        cpf = pltpu.async_copy(
            feat_hbm.at[pl.ds(base, nrows), pl.ds(col0, HF)],
            fbuf.at[b, pl.ds(0, nrows)], semf)
        cpl = pltpu.async_copy(lab_hbm.at[pl.ds(base, nrows)],
                               lbuf.at[b, pl.ds(0, nrows)], seml)
        cp1 = pltpu.async_copy(lab1_hbm.at[pl.ds(base, nrows)],
                               l1buf.at[b, pl.ds(0, nrows)], sem1)
        return cpf, cpl, cp1

    def process(nrows, b, copies):
        for cp in copies:
            cp.wait()

        @pl.loop(0, nrows // 16)
        def _grp(j):
            r0 = j * 16
            lab_vec = lbuf[b, pl.ds(r0, 16)]
            l1_vec = l1buf[b, pl.ds(r0, 16)]
            labp_vec = jnp.where(l1_vec <= 0, lab_vec, CPAD - 1)

            def load_row(l):
                return [fbuf[b, r0 + l, pl.ds(k * 16, 16)]
                        for k in range(HF // 16)]

            vals = load_row(0)
            labp = labp_vec[0]
            for l in range(16):
                if l + 1 < 16:
                    nvals = load_row(l + 1)
                    nlabp = labp_vec[l + 1]
                for k in range(HF // 16):
                    plsc.addupdate(acc.at[labp, pl.ds(k * 16, 16)], vals[k])
                plsc.addupdate(cnt.at[labp, :], e0)
                if l + 1 < 16:
                    vals, labp = nvals, nlabp

    def chunk_base(i):
        return (i * NPAIRS + pair) * CHUNK

    copies = issue(chunk_base(0), CHUNK, 0)
    for i in range(NCHUNKS):
        if i + 1 < NCHUNKS:
            nxt = issue(chunk_base(i + 1), CHUNK, (i + 1) % 2)
        process(CHUNK, i % 2, copies)
        if i + 1 < NCHUNKS:
            copies = nxt

    tail_base = MAIN_ROWS + pair * TAIL_A

    @pl.when(pair < NPAIRS - 1)
    def _tail_a():
        process(TAIL_A, 0, issue(tail_base, TAIL_A, 0))

    @pl.when(pair == NPAIRS - 1)
    def _tail_b():
        process(TAIL_B, 0, issue(MAIN_ROWS + (NPAIRS - 1) * TAIL_A, TAIL_B, 0))

    pltpu.async_copy(acc, outg_hbm.at[wid], semf0).wait()
    pltpu.async_copy(cnt, outc_hbm.at[wid], semf0).wait()


def _tc_finish_body(pg_ref, pc_ref, in_ref, tgt_ref, w_ref, out_ref):
    cnt = jnp.sum(pc_ref[...], axis=0)
    nums = jnp.sum(cnt, axis=1, keepdims=True) * 0.5
    denom = jnp.where(nums > 0, nums, 1.0)
    inv = 1.0 / denom

    pg = jnp.sum(pg_ref[...], axis=0)
    g0 = pg[0] * inv
    g1 = pg[1] * inv
    dn = (((1,), (1,)), ((), ()))
    vec = (lax.dot_general(in_ref[:, :HF], g0, dn,
                           precision=lax.Precision.HIGHEST)
           + lax.dot_general(in_ref[:, HF:], g1, dn,
                             precision=lax.Precision.HIGHEST)) * (1.0 / TEMP)

    ones_col = jnp.ones((B, 1), jnp.float32)
    nums_b = lax.dot_general(ones_col, nums, dn,
                             precision=lax.Precision.HIGHEST)
    col_id = lax.broadcasted_iota(jnp.int32, (B, CPAD), 1)
    m = jnp.logical_and(col_id < NC, nums_b > 0)
    mf = m.astype(jnp.float32)

    vecm = jnp.where(m, vec, 0.0)
    exps = jnp.exp(vecm) * mf
    sums = jnp.sum(exps, axis=1, keepdims=True) + 1e-6
    masked_sim = exps / sums
    log_probs = jnp.log(masked_sim + 1e-6)
    lossmat = -tgt_ref[...] * log_probs * w_ref[...]
    total = jnp.sum(lossmat) * (1.0 / B)
    out_ref[...] = jnp.full((8, 128), total, jnp.float32)


_tc_finish = pl.pallas_call(
    _tc_finish_body,
    out_shape=jax.ShapeDtypeStruct((8, 128), jnp.float32),
)


@jax.jit
def kernel(inputs, indexes, targets, weight, features, labels, labels_1):
    del indexes
    partg, partc = _sc_segment_sum(features, labels.astype(jnp.int32),
                                   labels_1.astype(jnp.int32))
    partg = partg.reshape(NPAIRS, 2, CPAD, HF)
    tgt = jnp.pad(targets, ((0, 0), (0, CPAD - NC)))
    out = _tc_finish(partg, partc, inputs, tgt, weight.reshape(B, 1))
    return out[0, 0]


# speedup vs baseline: 6.6574x; 1.1209x over previous
import functools

import jax
import jax.numpy as jnp
from jax import lax
from jax.experimental import pallas as pl
from jax.experimental.pallas import tpu as pltpu
from jax.experimental.pallas import tpu_sc as plsc

NF = 128
HF = NF // 2
NS = 100000
NC = 500
CPAD = 512
B = 128
TEMP = 0.05
NTILES = 32
NPAIRS = 16
CHUNK = 384
NCHUNKS = 16
MAIN_ROWS = NPAIRS * NCHUNKS * CHUNK
TAIL_A = 112
TAIL_B = 16

_mesh = plsc.VectorSubcoreMesh(core_axis_name="c", subcore_axis_name="s")


@functools.partial(
    pl.kernel,
    out_type=[
        jax.ShapeDtypeStruct((NTILES, CPAD, HF), jnp.float32),
        jax.ShapeDtypeStruct((NTILES, CPAD, 16), jnp.float32),
    ],
    mesh=_mesh,
    compiler_params=pltpu.CompilerParams(needs_layout_passes=False,
                                         use_tc_tiling_on_sc=False),
    scratch_types=[
        pltpu.VMEM((CPAD, HF), jnp.float32),
        pltpu.VMEM((CPAD, 16), jnp.float32),
        pltpu.VMEM((2, CHUNK, HF), jnp.float32),
        pltpu.VMEM((2, CHUNK), jnp.int32),
        pltpu.VMEM((2, CHUNK), jnp.int32),
        pltpu.SemaphoreType.DMA,
        pltpu.SemaphoreType.DMA,
        pltpu.SemaphoreType.DMA,
        pltpu.SemaphoreType.DMA,
        pltpu.SemaphoreType.DMA,
        pltpu.SemaphoreType.DMA,
    ],
)
def _sc_segment_sum(feat_hbm, lab_hbm, lab1_hbm, outg_hbm, outc_hbm,
                    acc, cnt, fbuf, lbuf, l1buf,
                    semf0, seml0, sem10, semf1, seml1, sem11):
    wid = lax.axis_index("s") * 2 + lax.axis_index("c")
    pair = wid // 2
    half = wid % 2
    col0 = half * HF
    zero16 = jnp.zeros((16,), jnp.float32)
    e0 = (lax.iota(jnp.int32, 16) == 0).astype(jnp.float32)
    sems = [(semf0, seml0, sem10), (semf1, seml1, sem11)]

    @pl.loop(0, CPAD)
    def _zero(r):
        for k in range(HF // 16):
            acc[r, pl.ds(k * 16, 16)] = zero16
        cnt[r, :] = zero16

    def issue(base, nrows, b):
        semf, seml, sem1 = sems[b]
        pltpu.async_copy(
            feat_hbm.at[pl.ds(base, nrows), pl.ds(col0, HF)],
            fbuf.at[b, pl.ds(0, nrows)], semf)
        pltpu.async_copy(lab_hbm.at[pl.ds(base, nrows)],
                         lbuf.at[b, pl.ds(0, nrows)], seml)
        pltpu.async_copy(lab1_hbm.at[pl.ds(base, nrows)],
                         l1buf.at[b, pl.ds(0, nrows)], sem1)

    def wait_buf(nrows, b):
        semf, seml, sem1 = sems[b]
        pltpu.make_async_copy(
            feat_hbm.at[pl.ds(0, nrows), pl.ds(col0, HF)],
            fbuf.at[b, pl.ds(0, nrows)], semf).wait()
        pltpu.make_async_copy(lab_hbm.at[pl.ds(0, nrows)],
                              lbuf.at[b, pl.ds(0, nrows)], seml).wait()
        pltpu.make_async_copy(lab1_hbm.at[pl.ds(0, nrows)],
                              l1buf.at[b, pl.ds(0, nrows)], sem1).wait()

    lane_iota = lax.iota(jnp.int32, 16)
    ones16 = jnp.ones((16,), jnp.float32)

    def process(nrows, b):
        @plsc.parallel_loop(0, nrows // 16, unroll=2)
        def _grp(j):
            r0 = j * 16
            lab_vec = lbuf[b, pl.ds(r0, 16)]
            l1_vec = l1buf[b, pl.ds(r0, 16)]
            labp_vec = jnp.where(l1_vec <= 0, lab_vec, CPAD - 1)
            plsc.addupdate_scatter(cnt, [labp_vec, lane_iota], ones16)

            def load_row(l):
                return [fbuf[b, r0 + l, pl.ds(k * 16, 16)]
                        for k in range(HF // 16)]

            vals = load_row(0)
            labp = labp_vec[0]
            for l in range(16):
                if l + 1 < 16:
                    nvals = load_row(l + 1)
                    nlabp = labp_vec[l + 1]
                for k in range(HF // 16):
                    plsc.addupdate(acc.at[labp, pl.ds(k * 16, 16)], vals[k])
                if l + 1 < 16:
                    vals, labp = nvals, nlabp

    def chunk_base(i):
        return (i * NPAIRS + pair) * CHUNK

    issue(chunk_base(0), CHUNK, 0)
    issue(chunk_base(1), CHUNK, 1)

    @pl.loop(0, NCHUNKS // 2)
    def _chunk_pair(it):
        c = it * 2
        wait_buf(CHUNK, 0)
        process(CHUNK, 0)

        @pl.when(c + 2 < NCHUNKS)
        def _issue0():
            issue(chunk_base(c + 2), CHUNK, 0)

        wait_buf(CHUNK, 1)
        process(CHUNK, 1)

        @pl.when(c + 3 < NCHUNKS)
        def _issue1():
            issue(chunk_base(c + 3), CHUNK, 1)

    tail_base = MAIN_ROWS + pair * TAIL_A

    @pl.when(pair < NPAIRS - 1)
    def _tail_a():
        issue(tail_base, TAIL_A, 0)
        wait_buf(TAIL_A, 0)
        process(TAIL_A, 0)

    @pl.when(pair == NPAIRS - 1)
    def _tail_b():
        issue(MAIN_ROWS + (NPAIRS - 1) * TAIL_A, TAIL_B, 0)
        wait_buf(TAIL_B, 0)
        process(TAIL_B, 0)

    out_idx = half * NPAIRS + pair
    pltpu.async_copy(acc, outg_hbm.at[out_idx], semf0).wait()
    pltpu.async_copy(cnt, outc_hbm.at[out_idx], semf0).wait()


def _tc_finish_body(pg_ref, pc_ref, in_ref, tgt_ref, w_ref, out_ref):
    cnt = jnp.sum(pc_ref[...], axis=0)
    nums = jnp.sum(cnt, axis=1, keepdims=True) * 0.5
    denom = jnp.where(nums > 0, nums, 1.0)
    inv = 1.0 / denom

    g0 = jnp.sum(pg_ref[:NPAIRS], axis=0) * inv
    g1 = jnp.sum(pg_ref[NPAIRS:], axis=0) * inv
    dn = (((1,), (1,)), ((), ()))
    vec = (lax.dot_general(in_ref[:, :HF], g0, dn,
                           precision=lax.Precision.HIGHEST)
           + lax.dot_general(in_ref[:, HF:], g1, dn,
                             precision=lax.Precision.HIGHEST)) * (1.0 / TEMP)

    ones_col = jnp.ones((B, 1), jnp.float32)
    nums_b = lax.dot_general(ones_col, nums, dn,
                             precision=lax.Precision.HIGHEST)
    col_id = lax.broadcasted_iota(jnp.int32, (B, CPAD), 1)
    m = jnp.logical_and(col_id < NC, nums_b > 0)
    mf = m.astype(jnp.float32)

    vecm = jnp.where(m, vec, 0.0)
    exps = jnp.exp(vecm) * mf
    sums = jnp.sum(exps, axis=1, keepdims=True) + 1e-6
    masked_sim = exps / sums
    log_probs = jnp.log(masked_sim + 1e-6)
    lossmat = -tgt_ref[...] * log_probs[:, :NC]
    rs = jnp.sum(lossmat, axis=1, keepdims=True)
    total = lax.dot_general(w_ref[...], rs, (((1,), (0,)), ((), ())),
                            precision=lax.Precision.HIGHEST)
    out_ref[...] = jnp.full((8, 128), total[0, 0] * (1.0 / B), jnp.float32)


_tc_finish = pl.pallas_call(
    _tc_finish_body,
    out_shape=jax.ShapeDtypeStruct((8, 128), jnp.float32),
)


@jax.jit
def kernel(inputs, indexes, targets, weight, features, labels, labels_1):
    del indexes
    partg, partc = _sc_segment_sum(features, labels.astype(jnp.int32),
                                   labels_1.astype(jnp.int32))
    out = _tc_finish(partg, partc, inputs, targets, weight.reshape(1, B))
    return out[0, 0]


# speedup vs baseline: 7.7648x; 1.1663x over previous
import functools

import jax
import jax.numpy as jnp
from jax import lax
from jax.experimental import pallas as pl
from jax.experimental.pallas import tpu as pltpu
from jax.experimental.pallas import tpu_sc as plsc

NF = 128
NS = 100000
NC = 500
CPAD = 512
B = 128
TEMP = 0.05
NTILES = 32
CHUNK = 128
NCHUNKS = 24
MAIN_ROWS = NTILES * NCHUNKS * CHUNK
TAIL_A = 80
TAIL_N = 21
TAIL_B = 16

_mesh = plsc.VectorSubcoreMesh(core_axis_name="c", subcore_axis_name="s")


@functools.partial(
    pl.kernel,
    out_type=[
        jax.ShapeDtypeStruct((NTILES, CPAD, NF), jnp.float32),
        jax.ShapeDtypeStruct((NTILES, CPAD // 8, NF), jnp.float32),
    ],
    mesh=_mesh,
    compiler_params=pltpu.CompilerParams(needs_layout_passes=False),
    scratch_types=[
        pltpu.VMEM((CPAD, NF), jnp.float32),
        pltpu.VMEM((CPAD // 8, NF), jnp.float32),
        pltpu.VMEM((2, CHUNK, NF), jnp.float32),
        pltpu.VMEM((2, CHUNK), jnp.int32),
        pltpu.VMEM((2, CHUNK), jnp.int32),
        pltpu.SemaphoreType.DMA,
        pltpu.SemaphoreType.DMA,
        pltpu.SemaphoreType.DMA,
        pltpu.SemaphoreType.DMA,
        pltpu.SemaphoreType.DMA,
        pltpu.SemaphoreType.DMA,
    ],
)
def _sc_segment_sum(feat_hbm, lab_hbm, lab1_hbm, outg_hbm, outc_hbm,
                    acc, cnt, fbuf, lbuf, l1buf,
                    semf0, seml0, sem10, semf1, seml1, sem11):
    wid = lax.axis_index("s") * 2 + lax.axis_index("c")
    zero16 = jnp.zeros((16,), jnp.float32)
    sems = [(semf0, seml0, sem10), (semf1, seml1, sem11)]
    lane_iota = lax.iota(jnp.int32, 16)
    ones16 = jnp.ones((16,), jnp.float32)

    @pl.loop(0, CPAD)
    def _zero(r):
        for k in range(NF // 16):
            acc[r, pl.ds(k * 16, 16)] = zero16

    @pl.loop(0, CPAD // 8)
    def _zeroc(r):
        for k in range(NF // 16):
            cnt[r, pl.ds(k * 16, 16)] = zero16

    def issue(base, nrows, b):
        semf, seml, sem1 = sems[b]
        pltpu.async_copy(feat_hbm.at[pl.ds(base, nrows)],
                         fbuf.at[b, pl.ds(0, nrows)], semf)
        pltpu.async_copy(lab_hbm.at[pl.ds(base, nrows)],
                         lbuf.at[b, pl.ds(0, nrows)], seml)
        pltpu.async_copy(lab1_hbm.at[pl.ds(base, nrows)],
                         l1buf.at[b, pl.ds(0, nrows)], sem1)

    def wait_buf(nrows, b):
        semf, seml, sem1 = sems[b]
        pltpu.make_async_copy(feat_hbm.at[pl.ds(0, nrows)],
                              fbuf.at[b, pl.ds(0, nrows)], semf).wait()
        pltpu.make_async_copy(lab_hbm.at[pl.ds(0, nrows)],
                              lbuf.at[b, pl.ds(0, nrows)], seml).wait()
        pltpu.make_async_copy(lab1_hbm.at[pl.ds(0, nrows)],
                              l1buf.at[b, pl.ds(0, nrows)], sem1).wait()

    def process(nrows, b):
        @plsc.parallel_loop(0, nrows // 16, unroll=2)
        def _grp(j):
            r0 = j * 16
            lab_vec = lbuf[b, pl.ds(r0, 16)]
            l1_vec = l1buf[b, pl.ds(r0, 16)]
            labp_vec = jnp.where(l1_vec <= 0, lab_vec, CPAD - 1)
            crow = lax.shift_right_logical(labp_vec, 3)
            ccol = jnp.bitwise_or(
                lax.shift_left(jnp.bitwise_and(labp_vec, 7), 4), lane_iota)
            plsc.addupdate_scatter(cnt, [crow, ccol], ones16)

            def load_row(l):
                return [fbuf[b, r0 + l, pl.ds(k * 16, 16)]
                        for k in range(NF // 16)]

            vals = load_row(0)
            labp = labp_vec[0]
            for l in range(16):
                if l + 1 < 16:
                    nvals = load_row(l + 1)
                    nlabp = labp_vec[l + 1]
                for k in range(NF // 16):
                    plsc.addupdate(acc.at[labp, pl.ds(k * 16, 16)], vals[k])
                if l + 1 < 16:
                    vals, labp = nvals, nlabp

    def chunk_base(i):
        return (i * NTILES + wid) * CHUNK

    issue(chunk_base(0), CHUNK, 0)
    issue(chunk_base(1), CHUNK, 1)

    @pl.loop(0, NCHUNKS // 2)
    def _chunk_pair(it):
        c = it * 2
        wait_buf(CHUNK, 0)
        process(CHUNK, 0)

        @pl.when(c + 2 < NCHUNKS)
        def _issue0():
            issue(chunk_base(c + 2), CHUNK, 0)

        wait_buf(CHUNK, 1)
        process(CHUNK, 1)

        @pl.when(c + 3 < NCHUNKS)
        def _issue1():
            issue(chunk_base(c + 3), CHUNK, 1)

    tail_base = MAIN_ROWS + wid * TAIL_A

    @pl.when(wid < TAIL_N)
    def _tail_a():
        issue(tail_base, TAIL_A, 0)
        wait_buf(TAIL_A, 0)
        process(TAIL_A, 0)

    @pl.when(wid == TAIL_N)
    def _tail_b():
        issue(MAIN_ROWS + TAIL_N * TAIL_A, TAIL_B, 0)
        wait_buf(TAIL_B, 0)
        process(TAIL_B, 0)

    pltpu.async_copy(acc, outg_hbm.at[wid], semf0).wait()
    pltpu.async_copy(cnt, outc_hbm.at[wid], semf0).wait()


def _tc_finish_body(pg_ref, pc_ref, in_ref, tgt_ref, w_ref, out_ref):
    cnt2 = jnp.sum(pc_ref[...], axis=0)
    rsel = lax.shift_right_logical(
        lax.broadcasted_iota(jnp.int32, (CPAD, CPAD // 8), 0), 3)
    csel = lax.broadcasted_iota(jnp.int32, (CPAD, CPAD // 8), 1)
    psel = (rsel == csel).astype(jnp.float32)
    dn0 = (((1,), (0,)), ((), ()))
    t = lax.dot_general(psel, cnt2, dn0,
                        precision=lax.Precision.HIGHEST)
    colg = lax.shift_right_logical(
        lax.broadcasted_iota(jnp.int32, (CPAD, NF), 1), 4)
    cmod = jnp.bitwise_and(lax.broadcasted_iota(jnp.int32, (CPAD, NF), 0), 7)
    wsel = (colg == cmod).astype(jnp.float32)
    nums = jnp.sum(t * wsel, axis=1, keepdims=True)
    denom = jnp.where(nums > 0, nums, 1.0)
    inv = 1.0 / denom

    g = jnp.sum(pg_ref[...], axis=0) * inv
    dn = (((1,), (1,)), ((), ()))
    vec = lax.dot_general(in_ref[...], g, dn,
                          precision=lax.Precision.HIGHEST) * (1.0 / TEMP)

    ones_col = jnp.ones((B, 1), jnp.float32)
    nums_b = lax.dot_general(ones_col, nums, dn,
                             precision=lax.Precision.HIGHEST)
    col_id = lax.broadcasted_iota(jnp.int32, (B, CPAD), 1)
    m = jnp.logical_and(col_id < NC, nums_b > 0)
    mf = m.astype(jnp.float32)

    vecm = jnp.where(m, vec, 0.0)
    exps = jnp.exp(vecm) * mf
    sums = jnp.sum(exps, axis=1, keepdims=True) + 1e-6
    masked_sim = exps / sums
    log_probs = jnp.log(masked_sim + 1e-6)
    lossmat = -tgt_ref[...] * log_probs[:, :NC]
    rs = jnp.sum(lossmat, axis=1, keepdims=True)
    total = lax.dot_general(w_ref[...], rs, (((1,), (0,)), ((), ())),
                            precision=lax.Precision.HIGHEST)
    out_ref[...] = jnp.full((8, 128), total[0, 0] * (1.0 / B), jnp.float32)


_tc_finish = pl.pallas_call(
    _tc_finish_body,
    out_shape=jax.ShapeDtypeStruct((8, 128), jnp.float32),
)


@jax.jit
def kernel(inputs, indexes, targets, weight, features, labels, labels_1):
    del indexes
    partg, partc = _sc_segment_sum(features, labels.astype(jnp.int32),
                                   labels_1.astype(jnp.int32))
    out = _tc_finish(partg, partc, inputs, targets, weight.reshape(1, B))
    return out[0, 0]
